# baseline (speedup 1.0000x reference)
import functools

import jax
import jax.numpy as jnp
from jax import lax
from jax.experimental import pallas as pl
from jax.experimental.pallas import tpu as pltpu
from jax.experimental.pallas import tpu_sc as plsc

D_MODEL = 768
NUM_HEADS = 12
_F32_MIN = float(jnp.finfo(jnp.float32).min)

_NUM_CORES = 2
_NUM_SUBCORES = 16
_NUM_WORKERS = _NUM_CORES * _NUM_SUBCORES


def _make_sc_pipe(n_tokens: int, n_bias: int, chunk: int, nbuf: int,
                  delay: int):
    assert n_tokens % _NUM_WORKERS == 0
    per_w = n_tokens // _NUM_WORKERS
    assert per_w % chunk == 0
    n_chunks = per_w // chunk
    assert n_chunks % 2 == 0 and nbuf % 2 == 0
    assert n_bias % (_NUM_WORKERS * 16) == 0
    bias_per_w = n_bias // _NUM_WORKERS

    mesh = plsc.VectorSubcoreMesh(core_axis_name="c", subcore_axis_name="s")

    @functools.partial(
        pl.kernel,
        out_type=(
            jax.ShapeDtypeStruct((n_tokens, D_MODEL), jnp.float32),
            jax.ShapeDtypeStruct((n_tokens,), jnp.float32),
            jax.ShapeDtypeStruct((n_bias,), jnp.float32),
        ),
        mesh=mesh,
        scratch_types=[
            pltpu.VMEM((per_w,), jnp.int32),
            pltpu.VMEM((per_w,), jnp.int32),
            pltpu.VMEM((per_w,), jnp.float32),
            pltpu.VMEM((bias_per_w,), jnp.float32),
            pltpu.VMEM((nbuf * chunk, D_MODEL), jnp.float32),
            pltpu.SemaphoreType.DMA,
            pltpu.SemaphoreType.DMA,
            pltpu.SemaphoreType.DMA,
        ],
    )
    def sc_pipe(ids_hbm, mask_hbm, table_hbm, out_hbm, ext_hbm, bias_hbm,
                idx_v, mask_v, ext_v, bias_v, bufs, gsem, ssem, msem):
        wid = lax.axis_index("s") * _NUM_CORES + lax.axis_index("c")
        base = wid * per_w
        n_pairs = n_chunks // 2
        pltpu.sync_copy(ids_hbm.at[pl.ds(base, per_w)], idx_v)
        gathers = [None] * n_chunks
        scatters = [None] * n_pairs
        for c in range(min(nbuf, n_chunks)):
            gathers[c] = pltpu.async_copy(
                table_hbm.at[idx_v.at[pl.ds(c * chunk, chunk)]],
                bufs.at[pl.ds((c % nbuf) * chunk, chunk)], gsem,
            )
        pltpu.sync_copy(mask_hbm.at[pl.ds(base, per_w)], mask_v)

        for i in range(per_w // 16):
            m = mask_v[pl.ds(i * 16, 16)].astype(jnp.float32)
            ext_v[pl.ds(i * 16, 16)] = (1.0 - m) * _F32_MIN
        ext_copy = pltpu.async_copy(ext_v, ext_hbm.at[pl.ds(base, per_w)], msem)

        zero16 = jnp.zeros((16,), jnp.float32)

        def zbody(i, carry):
            bias_v[pl.ds(i * 16, 16)] = zero16
            return carry

        lax.fori_loop(0, bias_per_w // 16, zbody, 0)
        bias_copy = pltpu.async_copy(
            bias_v, bias_hbm.at[pl.ds(wid * bias_per_w, bias_per_w)], msem
        )

        for p in range(n_pairs):
            c0 = 2 * p
            gathers[c0].wait()
            gathers[c0 + 1].wait()
            scatters[p] = pltpu.async_copy(
                bufs.at[pl.ds((c0 % nbuf) * chunk, 2 * chunk)],
                out_hbm.at[pl.ds(base + c0 * chunk, 2 * chunk)], ssem,
            )
            r = p - delay
            if 0 <= r and 2 * r + nbuf + 1 < n_chunks:
                scatters[r].wait()
                scatters[r] = None
                for k in range(2):
                    cc = 2 * r + nbuf + k
                    gathers[cc] = pltpu.async_copy(
                        table_hbm.at[idx_v.at[pl.ds(cc * chunk, chunk)]],
                        bufs.at[pl.ds((cc % nbuf) * chunk, chunk)], gsem,
                    )
        for s in scatters:
            if s is not None:
                s.wait()
        ext_copy.wait()
        bias_copy.wait()

    return sc_pipe


def kernel(encoder_input_ids, encoder_attention_mask, decoder_input_ids,
           decoder_attention_mask, embedding_table):
    batch, seq = encoder_input_ids.shape
    n_tokens = batch * seq
    n_bias = batch * NUM_HEADS * seq

    ids_flat = encoder_input_ids.reshape(n_tokens)
    mask_flat = encoder_attention_mask.reshape(n_tokens)

    pipe = _make_sc_pipe(n_tokens, n_bias, chunk=32, nbuf=4, delay=1)
    hidden, ext, bias = pipe(ids_flat, mask_flat, embedding_table)

    hidden = hidden.reshape(batch, seq, D_MODEL)
    ext = ext.reshape(batch, 1, 1, seq)
    bias = bias.reshape(batch, NUM_HEADS, seq, 1)

    return (hidden, ext, bias, decoder_input_ids, decoder_attention_mask,
            encoder_attention_mask)


# speedup vs baseline: 1.0509x; 1.0509x over previous
import functools

import jax
import jax.numpy as jnp
from jax import lax
from jax.experimental import pallas as pl
from jax.experimental.pallas import tpu as pltpu
from jax.experimental.pallas import tpu_sc as plsc

D_MODEL = 768
NUM_HEADS = 12
_F32_MIN = float(jnp.finfo(jnp.float32).min)

_NUM_CORES = 2
_NUM_SUBCORES = 16
_NUM_WORKERS = _NUM_CORES * _NUM_SUBCORES


def _make_sc_pipe(n_tokens: int, n_bias: int, chunk: int, nbuf: int,
                  delay: int):
    assert n_tokens % _NUM_WORKERS == 0
    per_w = n_tokens // _NUM_WORKERS
    assert per_w % chunk == 0
    n_chunks = per_w // chunk
    assert n_bias % (_NUM_WORKERS * 16) == 0
    bias_per_w = n_bias // _NUM_WORKERS

    mesh = plsc.VectorSubcoreMesh(core_axis_name="c", subcore_axis_name="s")

    @functools.partial(
        pl.kernel,
        out_type=(
            jax.ShapeDtypeStruct((n_tokens, D_MODEL), jnp.float32),
            jax.ShapeDtypeStruct((n_tokens,), jnp.float32),
            jax.ShapeDtypeStruct((n_bias,), jnp.float32),
        ),
        mesh=mesh,
        scratch_types=[
            pltpu.VMEM((per_w,), jnp.int32),
            pltpu.VMEM((per_w,), jnp.int32),
            pltpu.VMEM((per_w,), jnp.float32),
            pltpu.VMEM((bias_per_w,), jnp.float32),
            pltpu.VMEM((nbuf, chunk, D_MODEL), jnp.float32),
            pltpu.SemaphoreType.DMA,
            pltpu.SemaphoreType.DMA,
            pltpu.SemaphoreType.DMA,
        ],
    )
    def sc_pipe(ids_hbm, mask_hbm, table_hbm, out_hbm, ext_hbm, bias_hbm,
                idx_v, mask_v, ext_v, bias_v, bufs, gsem, ssem, msem):
        wid = lax.axis_index("s") * _NUM_CORES + lax.axis_index("c")
        base = wid * per_w
        pltpu.sync_copy(ids_hbm.at[pl.ds(base, per_w)], idx_v)
        gathers = [None] * n_chunks
        scatters = [None] * n_chunks
        for c in range(min(nbuf, n_chunks)):
            gathers[c] = pltpu.async_copy(
                table_hbm.at[idx_v.at[pl.ds(c * chunk, chunk)]], bufs.at[c], gsem
            )
        pltpu.sync_copy(mask_hbm.at[pl.ds(base, per_w)], mask_v)

        for i in range(per_w // 16):
            m = mask_v[pl.ds(i * 16, 16)].astype(jnp.float32)
            ext_v[pl.ds(i * 16, 16)] = (1.0 - m) * _F32_MIN
        ext_copy = pltpu.async_copy(ext_v, ext_hbm.at[pl.ds(base, per_w)], msem)

        zero16 = jnp.zeros((16,), jnp.float32)

        def zbody(i, carry):
            bias_v[pl.ds(i * 16, 16)] = zero16
            return carry

        lax.fori_loop(0, bias_per_w // 16, zbody, 0)
        bias_copy = pltpu.async_copy(
            bias_v, bias_hbm.at[pl.ds(wid * bias_per_w, bias_per_w)], msem
        )

        for c in range(n_chunks):
            gathers[c].wait()
            scatters[c] = pltpu.async_copy(
                bufs.at[c % nbuf], out_hbm.at[pl.ds(base + c * chunk, chunk)],
                ssem,
            )
            r = c - delay
            if 0 <= r and r + nbuf < n_chunks:
                scatters[r].wait()
                scatters[r] = None
                gathers[r + nbuf] = pltpu.async_copy(
                    table_hbm.at[idx_v.at[pl.ds((r + nbuf) * chunk, chunk)]],
                    bufs.at[r % nbuf], gsem,
                )
        for s in scatters:
            if s is not None:
                s.wait()
        ext_copy.wait()
        bias_copy.wait()

    return sc_pipe


def kernel(encoder_input_ids, encoder_attention_mask, decoder_input_ids,
           decoder_attention_mask, embedding_table):
    batch, seq = encoder_input_ids.shape
    n_tokens = batch * seq
    n_bias = batch * NUM_HEADS * seq

    ids_flat = encoder_input_ids.reshape(n_tokens)
    mask_flat = encoder_attention_mask.reshape(n_tokens)

    pipe = _make_sc_pipe(n_tokens, n_bias, chunk=32, nbuf=5, delay=2)
    hidden, ext, bias = pipe(ids_flat, mask_flat, embedding_table)

    hidden = hidden.reshape(batch, seq, D_MODEL)
    ext = ext.reshape(batch, 1, 1, seq)
    bias = bias.reshape(batch, NUM_HEADS, seq, 1)

    return (hidden, ext, bias, decoder_input_ids, decoder_attention_mask,
            encoder_attention_mask)
